# baseline (speedup 1.0000x reference)
import functools

import jax
import jax.numpy as jnp
from jax import lax
from jax.experimental import pallas as pl
from jax.experimental.pallas import tpu as pltpu
from jax.experimental.pallas import tpu_sc as plsc

BATCH = 16384
EMBED_DIM = 64
NUM_IDS = 1000000
NUM_TC = (NUM_IDS + 127) // 128
TC_PER_W = 245
WIN = 128
ID_SENTINEL = 1 << 30
TC_SENTINEL = 1 << 20
RING = 64
NBUF = 4


@functools.cache
def _build_lookup():
    info = plsc.get_sparse_core_info()
    num_workers = info.num_cores * info.num_subcores
    mesh = plsc.VectorSubcoreMesh(core_axis_name="c", subcore_axis_name="s")

    @functools.partial(
        pl.kernel,
        mesh=mesh,
        out_type=(
            jax.ShapeDtypeStruct((BATCH, EMBED_DIM), jnp.float32),
            jax.ShapeDtypeStruct((BATCH, EMBED_DIM), jnp.float32),
        ),
        scratch_types=[
            pltpu.VMEM((BATCH + 64,), jnp.int32),
            pltpu.VMEM((BATCH + 64,), jnp.int32),
            pltpu.VMEM((BATCH + 64,), jnp.int32),
            pltpu.VMEM((64,), jnp.int32),
            pltpu.VMEM((64,), jnp.int32),
            pltpu.VMEM((EMBED_DIM, WIN), jnp.float32),
            pltpu.VMEM((EMBED_DIM, WIN), jnp.float32),
            pltpu.VMEM((EMBED_DIM, WIN), jnp.float32),
            pltpu.VMEM((EMBED_DIM, WIN), jnp.float32),
            pltpu.VMEM((RING, EMBED_DIM), jnp.float32),
            pltpu.SemaphoreType.DMA,
            pltpu.SemaphoreType.DMA,
            pltpu.SemaphoreType.DMA,
            pltpu.SemaphoreType.DMA,
            pltpu.SemaphoreType.DMA,
        ],
        compiler_params=pltpu.CompilerParams(needs_layout_passes=False),
    )
    def lookup(usid_hbm, uord_hbm, uwin_hbm, ustarts_hbm, uwstarts_hbm,
               isid_hbm, iord_hbm, iwin_hbm, istarts_hbm, iwstarts_hbm,
               utab_hbm, itab_hbm, uout_hbm, iout_hbm,
               sid_v, ord_v, win_v, starts_v, wstarts_v,
               bufa, bufb, bufc, bufd, ring,
               sem0, sem1, sem2, sem3, wsem):
        wid = lax.axis_index("s") * info.num_cores + lax.axis_index("c")
        lanes = lax.iota(jnp.int32, 16)
        sems = [sem0, sem1, sem2, sem3]
        bufs = [bufa, bufb, bufc, bufd]
        widvec = jnp.broadcast_to(wid, (16,))

        def run_table(tab, sid_hbm, ord_hbm, winl_hbm, starts_hbm,
                      wstarts_hbm, out_hbm):
            pltpu.sync_copy(starts_hbm, starts_v)
            pltpu.sync_copy(wstarts_hbm, wstarts_v)
            pltpu.sync_copy(sid_hbm, sid_v)
            pltpu.sync_copy(ord_hbm, ord_v)
            pltpu.sync_copy(winl_hbm, win_v)
            p0 = plsc.load_gather(starts_v, [widvec])[0]
            q_lo = plsc.load_gather(wstarts_v, [widvec])[0]
            q_hi = plsc.load_gather(wstarts_v, [widvec + 1])[0]

            def win_tc(q):
                return jnp.minimum(win_v[pl.ds(q, 16)][0], NUM_TC - 1)

            def fetch(q, b):
                start = win_tc(q) * WIN
                pltpu.async_copy(tab.at[:, pl.ds(start, WIN)],
                                 bufs[b], sems[b])

            def wait_buf(q, b):
                pltpu.make_async_copy(tab.at[:, pl.ds(0, WIN)],
                                      bufs[b], sems[b]).wait()

            def drain_one(i, x):
                pltpu.make_async_copy(ring.at[pl.ds(0, 1)],
                                      out_hbm.at[pl.ds(0, 1)], wsem).wait()
                return x

            def proc_window(q, b, carry):
                win_start = win_tc(q) * WIN

                def cond(c):
                    return c[3]

                def body(c):
                    p, fired, prev, _ = c
                    svec = sid_v[pl.ds(p, 16)]
                    jvec = ord_v[pl.ds(p, 16)]
                    m = (svec >= win_start) & (svec < win_start + WIN)
                    cnt = plsc.all_reduce_population_count(m)[0]
                    for l in range(16):
                        @pl.when(l < cnt)
                        def _():
                            col = jnp.broadcast_to(svec[l] - win_start, (16,))
                            slot = lax.rem(fired + l, RING)
                            srow = jnp.broadcast_to(slot, (16,))
                            for mm in range(4):
                                rows = lanes + 16 * mm
                                g = plsc.load_gather(bufs[b], [rows, col])
                                plsc.store_scatter(ring, [srow, rows], g)
                            pltpu.async_copy(
                                ring.at[pl.ds(slot, 1)],
                                out_hbm.at[pl.ds(jvec[l], 1)], wsem)
                    lax.fori_loop(0, prev, drain_one, 0)
                    return (p + cnt, fired + cnt, cnt, cnt == 16)

                return lax.while_loop(cond, body, carry[:3] + (True,))[:3]

            for b in range(NBUF):
                fetch(q_lo + b, b)
            carry = (p0, jnp.int32(0), jnp.int32(0))

            def t_body(t, carry):
                q = q_lo + t * NBUF
                for b in range(NBUF):
                    wait_buf(q + b, b)
                    carry = proc_window(q + b, b, carry)
                    fetch(q + b + NBUF, b)
                return carry

            n_groups = lax.div(q_hi - q_lo + (NBUF - 1), jnp.int32(NBUF))
            carry = lax.fori_loop(0, n_groups, t_body, carry)
            for b in range(NBUF):
                wait_buf(0, b)
            lax.fori_loop(0, carry[2], drain_one, 0)

        run_table(utab_hbm, usid_hbm, uord_hbm, uwin_hbm, ustarts_hbm,
                  uwstarts_hbm, uout_hbm)
        run_table(itab_hbm, isid_hbm, iord_hbm, iwin_hbm, istarts_hbm,
                  iwstarts_hbm, iout_hbm)

    return lookup


def _prep(ids):
    ids = ids.astype(jnp.int32)
    sid, order = lax.sort(
        (ids, jnp.arange(BATCH, dtype=jnp.int32)), num_keys=1)
    wtc = sid >> 7
    first = jnp.concatenate(
        [jnp.ones((1,), jnp.bool_), wtc[1:] != wtc[:-1]])
    wins = jnp.sort(jnp.where(first, wtc, TC_SENTINEL))
    bounds = jnp.minimum(jnp.arange(33, dtype=jnp.int32) * TC_PER_W, NUM_TC)
    wstarts = jnp.searchsorted(wins, bounds).astype(jnp.int32)
    starts = jnp.searchsorted(sid, bounds * WIN).astype(jnp.int32)
    pad64 = jnp.full((64,), ID_SENTINEL, jnp.int32)
    return (jnp.concatenate([sid, pad64]),
            jnp.concatenate([order, jnp.zeros((64,), jnp.int32)]),
            jnp.concatenate([wins, jnp.full((64,), TC_SENTINEL, jnp.int32)]),
            jnp.pad(starts, (0, 31)),
            jnp.pad(wstarts, (0, 31)))


def kernel(user_ids, item_ids, user_table, item_table):
    lookup = _build_lookup()
    up = _prep(user_ids)
    ip = _prep(item_ids)
    return lookup(*up, *ip, user_table.T, item_table.T)


# speedup vs baseline: 1.4787x; 1.4787x over previous
import functools

import jax
import jax.numpy as jnp
from jax import lax
from jax.experimental import pallas as pl
from jax.experimental.pallas import tpu as pltpu
from jax.experimental.pallas import tpu_sc as plsc

BATCH = 16384
EMBED_DIM = 64
NUM_IDS = 1000000
IDS_PER_W = 31232
WIN = 512
TAIL = NUM_IDS - 32 * IDS_PER_W
SENTINEL = 1 << 30
RING = 64


@functools.cache
def _build_lookup():
    info = plsc.get_sparse_core_info()
    num_workers = info.num_cores * info.num_subcores
    mesh = plsc.VectorSubcoreMesh(core_axis_name="c", subcore_axis_name="s")

    @functools.partial(
        pl.kernel,
        mesh=mesh,
        out_type=(
            jax.ShapeDtypeStruct((BATCH, EMBED_DIM), jnp.float32),
            jax.ShapeDtypeStruct((BATCH, EMBED_DIM), jnp.float32),
        ),
        scratch_types=[
            pltpu.VMEM((BATCH + 64,), jnp.int32),
            pltpu.VMEM((BATCH + 64,), jnp.int32),
            pltpu.VMEM((64,), jnp.int32),
            pltpu.VMEM((EMBED_DIM, WIN), jnp.float32),
            pltpu.VMEM((EMBED_DIM, WIN), jnp.float32),
            pltpu.VMEM((EMBED_DIM, 128), jnp.float32),
            pltpu.VMEM((RING, EMBED_DIM), jnp.float32),
            pltpu.SemaphoreType.DMA,
            pltpu.SemaphoreType.DMA,
            pltpu.SemaphoreType.DMA,
            pltpu.SemaphoreType.DMA,
            pltpu.SemaphoreType.DMA,
        ],
        compiler_params=pltpu.CompilerParams(needs_layout_passes=False),
    )
    def lookup(usid_hbm, uord_hbm, ustarts_hbm, isid_hbm, iord_hbm,
               istarts_hbm, utab_hbm, itab_hbm, uout_hbm, iout_hbm,
               sid_v, ord_v, starts_v, buf0, buf1, buf2, ring,
               sem0, sem0b, sem1, sem1b, wsem):
        wid = lax.axis_index("s") * info.num_cores + lax.axis_index("c")
        base_id = wid * IDS_PER_W
        lanes = lax.iota(jnp.int32, 16)

        def win_fetch(tab, start, buf, sa, sb):
            for i, s in zip(range(4), (sa, sb, sa, sb)):
                pltpu.async_copy(
                    tab.at[pl.ds(16 * i, 16), pl.ds(start, WIN)],
                    buf.at[pl.ds(16 * i, 16)], s)

        def win_wait(tab, buf, sa, sb):
            for i, s in zip(range(4), (sa, sb, sa, sb)):
                pltpu.make_async_copy(
                    tab.at[pl.ds(16 * i, 16), pl.ds(0, WIN)],
                    buf.at[pl.ds(16 * i, 16)], s).wait()

        def run_table(tab, sid_hbm, ord_hbm, starts_hbm, out_hbm):
            pltpu.sync_copy(starts_hbm, starts_v)
            pltpu.sync_copy(sid_hbm, sid_v)
            pltpu.sync_copy(ord_hbm, ord_v)
            p0 = plsc.load_gather(
                starts_v, [jnp.broadcast_to(wid, (16,))])[0]

            def drain_one(i, x):
                pltpu.make_async_copy(ring.at[pl.ds(0, 1)],
                                      out_hbm.at[pl.ds(0, 1)], wsem).wait()
                return x

            def proc_window(buf, win_start, width, carry):
                def cond(c):
                    return c[3]

                def body(c):
                    p, fired, prev, _ = c
                    svec = sid_v[pl.ds(p, 16)]
                    jvec = ord_v[pl.ds(p, 16)]
                    m = (svec >= win_start) & (svec < win_start + width)
                    cnt = plsc.all_reduce_population_count(m)[0]
                    for l in range(16):
                        @pl.when(l < cnt)
                        def _():
                            col = jnp.broadcast_to(svec[l] - win_start, (16,))
                            slot = lax.rem(fired + l, RING)
                            srow = jnp.broadcast_to(slot, (16,))
                            for mm in range(4):
                                rows = lanes + 16 * mm
                                g = plsc.load_gather(buf, [rows, col])
                                plsc.store_scatter(ring, [srow, rows], g)
                            pltpu.async_copy(
                                ring.at[pl.ds(slot, 1)],
                                out_hbm.at[pl.ds(jvec[l], 1)], wsem)
                    lax.fori_loop(0, prev, drain_one, 0)
                    return (p + cnt, fired + cnt, cnt, cnt == 16)

                return lax.while_loop(cond, body, carry[:3] + (True,))[:3]

            win_fetch(tab, base_id, buf0, sem0, sem0b)
            carry = (p0, jnp.int32(0), jnp.int32(0))

            def t_body(t, carry):
                w1 = base_id + (2 * t + 1) * WIN
                win_fetch(tab, w1, buf1, sem1, sem1b)
                win_wait(tab, buf0, sem0, sem0b)
                carry = proc_window(buf0, base_id + 2 * t * WIN, WIN, carry)

                @pl.when(t < 30)
                def _():
                    win_fetch(tab, base_id + (2 * t + 2) * WIN, buf0,
                              sem0, sem0b)
                win_wait(tab, buf1, sem1, sem1b)
                carry = proc_window(buf1, w1, WIN, carry)
                return carry

            carry = lax.fori_loop(0, 31, t_body, carry)
            tail_start = base_id + 62 * WIN
            pltpu.sync_copy(tab.at[:, pl.ds(tail_start, 128)], buf2)
            carry = proc_window(buf2, tail_start, 128, carry)
            lax.fori_loop(0, carry[2], drain_one, 0)

        run_table(utab_hbm, usid_hbm, uord_hbm, ustarts_hbm, uout_hbm)
        run_table(itab_hbm, isid_hbm, iord_hbm, istarts_hbm, iout_hbm)

    return lookup


def _prep(ids):
    ids = ids.astype(jnp.int32)
    sid, order = lax.sort(
        (ids, jnp.arange(BATCH, dtype=jnp.int32)), num_keys=1)
    bounds = jnp.arange(33, dtype=jnp.int32) * IDS_PER_W
    starts = jnp.searchsorted(sid, bounds).astype(jnp.int32)
    starts = jnp.pad(starts, (0, 31))
    pad = jnp.full((64,), SENTINEL, jnp.int32)
    return (jnp.concatenate([sid, pad]),
            jnp.concatenate([order, jnp.zeros((64,), jnp.int32)]),
            starts)


def kernel(user_ids, item_ids, user_table, item_table):
    lookup = _build_lookup()
    usid, uord, ustarts = _prep(user_ids)
    isid, iord, istarts = _prep(item_ids)
    return lookup(usid, uord, ustarts, isid, iord, istarts,
                  user_table.T, item_table.T)
